# baseline (speedup 1.0000x reference)
import functools

import jax
import jax.numpy as jnp
from jax import lax
from jax.experimental import pallas as pl
from jax.experimental.pallas import tpu as pltpu
from jax.experimental.pallas import tpu_sc as plsc

EMB = 64
NC = 2
NS = 16
NW = NC * NS
CHUNK = 512
VB = 8192


def _tc_transpose_table(table):
    V = table.shape[0]
    K = VB // 2
    grid = (V + VB - 1) // VB

    def body(x_ref, y_ref):
        x = x_ref[...]
        xt_l = x[:, :K].T
        eye = (jax.lax.broadcasted_iota(jnp.int32, (EMB, EMB), 0)
               == jax.lax.broadcasted_iota(jnp.int32, (EMB, EMB), 1)
               ).astype(jnp.float32)
        xt_r = jax.lax.dot_general(
            x[:, K:], eye, (((0,), (0,)), ((), ())),
            preferred_element_type=jnp.float32,
        )
        y_ref[...] = jnp.concatenate([xt_l, xt_r], axis=1)

    return pl.pallas_call(
        body,
        grid=(grid,),
        in_specs=[pl.BlockSpec((EMB, VB), lambda j: (0, j))],
        out_specs=pl.BlockSpec((K, 2 * EMB), lambda j: (j, 0)),
        out_shape=jax.ShapeDtypeStruct((grid * K, 2 * EMB), jnp.float32),
    )(table.T)


def _sc_gather_pair(table, idx_a, idx_b):
    n = idx_a.shape[0]
    per_w = n // NW
    n_chunks = per_w // CHUNK
    assert per_w % CHUNK == 0 and per_w % 512 == 0 and n % NW == 0

    mesh = plsc.VectorSubcoreMesh(core_axis_name="c", subcore_axis_name="s")

    @functools.partial(
        pl.kernel,
        out_type=(
            jax.ShapeDtypeStruct((n, EMB), jnp.float32),
            jax.ShapeDtypeStruct((n, EMB), jnp.float32),
        ),
        mesh=mesh,
        scratch_types=[
            pltpu.VMEM((per_w,), jnp.int32),
            pltpu.VMEM((per_w,), jnp.int32),
            pltpu.VMEM((per_w,), jnp.int32),
            pltpu.VMEM((per_w,), jnp.int32),
            pltpu.VMEM((CHUNK, EMB), jnp.float32),
            pltpu.VMEM((CHUNK, EMB), jnp.float32),
            pltpu.SemaphoreType.DMA,
            pltpu.SemaphoreType.DMA,
        ],
        compiler_params=pltpu.CompilerParams(
            use_tc_tiling_on_sc=False, needs_layout_passes=False
        ),
    )
    def gather_kernel(idx_a_hbm, idx_b_hbm, table_hbm, out_a_hbm, out_b_hbm,
                      raw_a_v, raw_b_v, idx_a_v, idx_b_v, buf0, buf1,
                      sem0, sem1):
        wid = lax.axis_index("s") * NC + lax.axis_index("c")
        base = wid * per_w
        pltpu.sync_copy(idx_a_hbm.at[pl.ds(base, per_w)], raw_a_v)
        pltpu.sync_copy(idx_b_hbm.at[pl.ds(base, per_w)], raw_b_v)

        lane = lax.iota(jnp.int32, 16)

        def permute_remap(raw_v, idx_v):
            def body(k, _):
                p = k * 16 + lane
                q = p & 511
                src = (p - q) + ((q & 1) << 8) + (q >> 1)
                v = plsc.load_gather(raw_v, [src])
                r = v & (VB - 1)
                i = (v - r) + ((r & (VB // 2 - 1)) << 1) + (r >> 12)
                idx_v[pl.ds(k * 16, 16)] = i
                return ()

            lax.fori_loop(0, per_w // 16, body, (), unroll=False)

        permute_remap(raw_a_v, idx_a_v)
        permute_remap(raw_b_v, idx_b_v)

        def one_array(idx_v, out_hbm):
            def body(c, _):
                off = c * CHUNK
                pltpu.async_copy(
                    table_hbm.at[idx_v.at[pl.ds(off, CHUNK)]], buf0, sem0
                ).wait()
                pltpu.sync_copy(buf0, out_hbm.at[pl.ds(base + off, CHUNK)])
                return ()

            lax.fori_loop(0, n_chunks, body, (), unroll=False)

        one_array(idx_a_v, out_a_hbm)
        one_array(idx_b_v, out_b_hbm)

    return gather_kernel(idx_a, idx_b, table)


def _tc_finalize(tmp_a, tmp_b, W, b, L, B):
    nb = B // 512

    def body(ta_ref, tb_ref, w_ref, b_ref,
             ea1_ref, ea2_ref, eb1_ref, eb2_ref,
             pa1_ref, pa2_ref, pb1_ref, pb2_ref):
        w = w_ref[...]
        bc = b_ref[...]

        def one(t_ref, e1_ref, e2_ref, p1_ref, p2_ref):
            xt = t_ref[0, 0].T
            e = jnp.concatenate([xt[:EMB], xt[EMB:]], axis=1)
            e1_ref[0] = e
            e2_ref[0] = e
            p = jnp.dot(w, jnp.maximum(e, 0.0),
                        preferred_element_type=jnp.float32) + bc
            p1_ref[0] = p
            p2_ref[0] = p

        one(ta_ref, ea1_ref, ea2_ref, pa1_ref, pa2_ref)
        one(tb_ref, eb1_ref, eb2_ref, pb1_ref, pb2_ref)

    in_blk = pl.BlockSpec((1, 1, 256, 128), lambda l, j: (l, j, 0, 0))
    out_blk = pl.BlockSpec((1, EMB, 512), lambda l, j: (l, 0, j))
    out_sh = jax.ShapeDtypeStruct((L, EMB, B), jnp.float32)
    return pl.pallas_call(
        body,
        grid=(L, nb),
        in_specs=[
            in_blk,
            in_blk,
            pl.BlockSpec((EMB, EMB), lambda l, j: (0, 0)),
            pl.BlockSpec((EMB, 1), lambda l, j: (0, 0)),
        ],
        out_specs=[out_blk] * 8,
        out_shape=(out_sh,) * 8,
    )(tmp_a, tmp_b, W, b)


def kernel(tensor_day, tensor_diagnoses, table, W, b):
    B, L = tensor_day.shape
    n = B * L
    idx_diag = tensor_diagnoses.T.reshape(n).astype(jnp.int32)
    idx_day = tensor_day.T.reshape(n).astype(jnp.int32)

    table2 = _tc_transpose_table(table)
    table_lin = table2.reshape(table2.shape[0] * 2, EMB)

    tmp_diag, tmp_day = _sc_gather_pair(table_lin, idx_diag, idx_day)
    tmp_diag = tmp_diag.reshape(L, B // 512, 256, 128)
    tmp_day = tmp_day.reshape(L, B // 512, 256, 128)

    (ed_x, ed_y, ey_x, ey_y, pd_x, pd_y, py_x, py_y) = _tc_finalize(
        tmp_diag, tmp_day, W, b.reshape(EMB, 1), L, B
    )

    def to_logical(x):
        return jnp.transpose(x, (2, 0, 1))

    return (
        (to_logical(ed_x), to_logical(ey_x)),
        (to_logical(pd_x), to_logical(py_x)),
        (to_logical(ed_y), to_logical(ey_y)),
        (to_logical(pd_y), to_logical(py_y)),
    )


# speedup vs baseline: 1.1381x; 1.1381x over previous
import functools

import jax
import jax.numpy as jnp
from jax import lax
from jax.experimental import pallas as pl
from jax.experimental.pallas import tpu as pltpu
from jax.experimental.pallas import tpu_sc as plsc

EMB = 64
NC = 2
NS = 16
NW = NC * NS
CHUNK = 512
VB = 8192


def _tc_transpose_table(table):
    V = table.shape[0]
    K = VB // 2
    grid = (V + VB - 1) // VB

    def body(x_ref, y_ref):
        x = x_ref[...]
        xt_l = x[:, :K].T
        eye = (jax.lax.broadcasted_iota(jnp.int32, (EMB, EMB), 0)
               == jax.lax.broadcasted_iota(jnp.int32, (EMB, EMB), 1)
               ).astype(jnp.float32)
        xt_r = jax.lax.dot_general(
            x[:, K:], eye, (((0,), (0,)), ((), ())),
            preferred_element_type=jnp.float32,
        )
        y_ref[...] = jnp.concatenate([xt_l, xt_r], axis=1)

    return pl.pallas_call(
        body,
        grid=(grid,),
        in_specs=[pl.BlockSpec((EMB, VB), lambda j: (0, j))],
        out_specs=pl.BlockSpec((K, 2 * EMB), lambda j: (j, 0)),
        out_shape=jax.ShapeDtypeStruct((grid * K, 2 * EMB), jnp.float32),
    )(table.T)


def _sc_gather_pair(table, idx_a, idx_b):
    n = idx_a.shape[0]
    per_w = n // NW
    n_chunks = per_w // CHUNK
    assert per_w % CHUNK == 0 and per_w % 512 == 0 and n % NW == 0

    mesh = plsc.VectorSubcoreMesh(core_axis_name="c", subcore_axis_name="s")

    @functools.partial(
        pl.kernel,
        out_type=(
            jax.ShapeDtypeStruct((n, EMB), jnp.float32),
            jax.ShapeDtypeStruct((n, EMB), jnp.float32),
        ),
        mesh=mesh,
        scratch_types=[
            pltpu.VMEM((per_w,), jnp.int32),
            pltpu.VMEM((per_w,), jnp.int32),
            pltpu.VMEM((per_w,), jnp.int32),
            pltpu.VMEM((per_w,), jnp.int32),
            pltpu.VMEM((CHUNK, EMB), jnp.float32),
            pltpu.VMEM((CHUNK, EMB), jnp.float32),
            pltpu.SemaphoreType.DMA,
            pltpu.SemaphoreType.DMA,
        ],
        compiler_params=pltpu.CompilerParams(
            use_tc_tiling_on_sc=False, needs_layout_passes=False
        ),
    )
    def gather_kernel(idx_a_hbm, idx_b_hbm, table_hbm, out_a_hbm, out_b_hbm,
                      raw_a_v, raw_b_v, idx_a_v, idx_b_v, buf0, buf1,
                      sem0, sem1):
        wid = lax.axis_index("s") * NC + lax.axis_index("c")
        base = wid * per_w
        pltpu.sync_copy(idx_a_hbm.at[pl.ds(base, per_w)], raw_a_v)
        pltpu.sync_copy(idx_b_hbm.at[pl.ds(base, per_w)], raw_b_v)

        lane = lax.iota(jnp.int32, 16)

        def permute_remap(raw_v, idx_v):
            def body(k, _):
                p = k * 16 + lane
                q = p & 511
                src = (p - q) + ((q & 1) << 8) + (q >> 1)
                v = plsc.load_gather(raw_v, [src])
                r = v & (VB - 1)
                i = (v - r) + ((r & (VB // 2 - 1)) << 1) + (r >> 12)
                idx_v[pl.ds(k * 16, 16)] = i
                return ()

            lax.fori_loop(0, per_w // 16, body, (), unroll=False)

        permute_remap(raw_a_v, idx_a_v)
        permute_remap(raw_b_v, idx_b_v)

        def one_array(idx_v, out_hbm):
            def body(c, _):
                off = c * CHUNK
                pltpu.async_copy(
                    table_hbm.at[idx_v.at[pl.ds(off, CHUNK)]], buf0, sem0
                ).wait()
                pltpu.sync_copy(buf0, out_hbm.at[pl.ds(base + off, CHUNK)])
                return ()

            lax.fori_loop(0, n_chunks, body, (), unroll=False)

        one_array(idx_a_v, out_a_hbm)
        one_array(idx_b_v, out_b_hbm)

    return gather_kernel(idx_a, idx_b, table)


def _tc_finalize(tmp_a, tmp_b, W, b, L, B):
    nb = B // 1024

    def body(ta_ref, tb_ref, w_ref, b_ref,
             ea1_ref, ea2_ref, eb1_ref, eb2_ref,
             pa1_ref, pa2_ref, pb1_ref, pb2_ref):
        w = w_ref[...]
        bc = b_ref[...]

        def one(t_ref, e1_ref, e2_ref, p1_ref, p2_ref):
            xt0 = t_ref[0, 0].T
            xt1 = t_ref[0, 1].T
            e = jnp.concatenate(
                [xt0[:EMB], xt0[EMB:], xt1[:EMB], xt1[EMB:]], axis=1
            )
            e1_ref[0] = e
            e2_ref[0] = e
            p = jnp.dot(w, jnp.maximum(e, 0.0),
                        preferred_element_type=jnp.float32) + bc
            p1_ref[0] = p
            p2_ref[0] = p

        one(ta_ref, ea1_ref, ea2_ref, pa1_ref, pa2_ref)
        one(tb_ref, eb1_ref, eb2_ref, pb1_ref, pb2_ref)

    in_blk = pl.BlockSpec((1, 2, 256, 128), lambda l, j: (l, j, 0, 0))
    out_blk = pl.BlockSpec((1, EMB, 1024), lambda l, j: (l, 0, j))
    out_sh = jax.ShapeDtypeStruct((L, EMB, B), jnp.float32)
    return pl.pallas_call(
        body,
        grid=(L, nb),
        in_specs=[
            in_blk,
            in_blk,
            pl.BlockSpec((EMB, EMB), lambda l, j: (0, 0)),
            pl.BlockSpec((EMB, 1), lambda l, j: (0, 0)),
        ],
        out_specs=[out_blk] * 8,
        out_shape=(out_sh,) * 8,
    )(tmp_a, tmp_b, W, b)


def kernel(tensor_day, tensor_diagnoses, table, W, b):
    B, L = tensor_day.shape
    n = B * L
    idx_diag = tensor_diagnoses.T.reshape(n).astype(jnp.int32)
    idx_day = tensor_day.T.reshape(n).astype(jnp.int32)

    table2 = _tc_transpose_table(table)
    table_lin = table2.reshape(table2.shape[0] * 2, EMB)

    tmp_diag, tmp_day = _sc_gather_pair(table_lin, idx_diag, idx_day)
    tmp_diag = tmp_diag.reshape(L, B // 512, 256, 128)
    tmp_day = tmp_day.reshape(L, B // 512, 256, 128)

    (ed_x, ed_y, ey_x, ey_y, pd_x, pd_y, py_x, py_y) = _tc_finalize(
        tmp_diag, tmp_day, W, b.reshape(EMB, 1), L, B
    )

    def to_logical(x):
        return jnp.transpose(x, (2, 0, 1))

    return (
        (to_logical(ed_x), to_logical(ey_x)),
        (to_logical(pd_x), to_logical(py_x)),
        (to_logical(ed_y), to_logical(ey_y)),
        (to_logical(pd_y), to_logical(py_y)),
    )


# speedup vs baseline: 1.2177x; 1.0699x over previous
import functools

import jax
import jax.numpy as jnp
from jax import lax
from jax.experimental import pallas as pl
from jax.experimental.pallas import tpu as pltpu
from jax.experimental.pallas import tpu_sc as plsc

EMB = 64
NC = 2
NS = 16
NW = NC * NS
CHUNK = 512
VB = 8192


def _tc_transpose_table(table):
    V = table.shape[0]
    K = VB // 2
    grid = (V + VB - 1) // VB

    def body(x_ref, y_ref):
        x = x_ref[...]
        xt_l = x[:, :K].T
        eye = (jax.lax.broadcasted_iota(jnp.int32, (EMB, EMB), 0)
               == jax.lax.broadcasted_iota(jnp.int32, (EMB, EMB), 1)
               ).astype(jnp.float32)
        xt_r = jax.lax.dot_general(
            x[:, K:], eye, (((0,), (0,)), ((), ())),
            preferred_element_type=jnp.float32,
        )
        y_ref[...] = jnp.concatenate([xt_l, xt_r], axis=1)

    return pl.pallas_call(
        body,
        grid=(grid,),
        in_specs=[pl.BlockSpec((EMB, VB), lambda j: (0, j))],
        out_specs=pl.BlockSpec((K, 2 * EMB), lambda j: (j, 0)),
        out_shape=jax.ShapeDtypeStruct((grid * K, 2 * EMB), jnp.float32),
    )(table.T)


def _sc_gather_pair(table, idx_a, idx_b):
    n = idx_a.shape[0]
    per_w = n // NW
    n_chunks = per_w // CHUNK
    assert per_w % CHUNK == 0 and per_w % 512 == 0 and n % NW == 0

    mesh = plsc.VectorSubcoreMesh(core_axis_name="c", subcore_axis_name="s")

    @functools.partial(
        pl.kernel,
        out_type=(
            jax.ShapeDtypeStruct((n, EMB), jnp.float32),
            jax.ShapeDtypeStruct((n, EMB), jnp.float32),
        ),
        mesh=mesh,
        scratch_types=[
            pltpu.VMEM((per_w,), jnp.int32),
            pltpu.VMEM((per_w,), jnp.int32),
            pltpu.VMEM((per_w,), jnp.int32),
            pltpu.VMEM((per_w,), jnp.int32),
            pltpu.VMEM((CHUNK, EMB), jnp.float32),
            pltpu.VMEM((CHUNK, EMB), jnp.float32),
            pltpu.SemaphoreType.DMA,
            pltpu.SemaphoreType.DMA,
        ],
        compiler_params=pltpu.CompilerParams(
            use_tc_tiling_on_sc=False, needs_layout_passes=False
        ),
    )
    def gather_kernel(idx_a_hbm, idx_b_hbm, table_hbm, out_a_hbm, out_b_hbm,
                      raw_a_v, raw_b_v, idx_a_v, idx_b_v, buf0, buf1,
                      sem0, sem1):
        wid = lax.axis_index("s") * NC + lax.axis_index("c")
        base = wid * per_w
        pltpu.sync_copy(idx_a_hbm.at[pl.ds(base, per_w)], raw_a_v)
        pltpu.sync_copy(idx_b_hbm.at[pl.ds(base, per_w)], raw_b_v)

        lane = lax.iota(jnp.int32, 16)

        def permute_remap(raw_v, idx_v):
            def body(k, _):
                p = k * 16 + lane
                q = p & 511
                src = (p - q) + ((q & 1) << 8) + (q >> 1)
                v = plsc.load_gather(raw_v, [src])
                r = v & (VB - 1)
                i = (v - r) + ((r & (VB // 2 - 1)) << 1) + (r >> 12)
                idx_v[pl.ds(k * 16, 16)] = i
                return ()

            lax.fori_loop(0, per_w // 16, body, (), unroll=False)

        permute_remap(raw_a_v, idx_a_v)
        permute_remap(raw_b_v, idx_b_v)

        def one_array(idx_v, out_hbm):
            def body(c, _):
                off = c * CHUNK
                pltpu.async_copy(
                    table_hbm.at[idx_v.at[pl.ds(off, CHUNK)]], buf0, sem0
                ).wait()
                pltpu.sync_copy(buf0, out_hbm.at[pl.ds(base + off, CHUNK)])
                return ()

            lax.fori_loop(0, n_chunks, body, (), unroll=False)

        one_array(idx_a_v, out_a_hbm)
        one_array(idx_b_v, out_b_hbm)

    return gather_kernel(idx_a, idx_b, table)


def _tc_finalize(tmp_a, tmp_b, W, b, L, B):
    nb = B // 2048

    def body(ta_ref, tb_ref, w_ref, b_ref,
             ea1_ref, ea2_ref, eb1_ref, eb2_ref,
             pa1_ref, pa2_ref, pb1_ref, pb2_ref):
        w = w_ref[...]
        bc = b_ref[...]

        def one(t_ref, e1_ref, e2_ref, p1_ref, p2_ref):
            halves = []
            for g in range(4):
                xt = t_ref[0, g].T
                halves.append(xt[:EMB])
                halves.append(xt[EMB:])
            e = jnp.concatenate(halves, axis=1)
            e1_ref[0] = e
            e2_ref[0] = e
            p = jnp.dot(w, jnp.maximum(e, 0.0),
                        preferred_element_type=jnp.float32) + bc
            p1_ref[0] = p
            p2_ref[0] = p

        one(ta_ref, ea1_ref, ea2_ref, pa1_ref, pa2_ref)
        one(tb_ref, eb1_ref, eb2_ref, pb1_ref, pb2_ref)

    in_blk = pl.BlockSpec((1, 4, 256, 128), lambda l, j: (l, j, 0, 0))
    out_blk = pl.BlockSpec((1, EMB, 2048), lambda l, j: (l, 0, j))
    out_sh = jax.ShapeDtypeStruct((L, EMB, B), jnp.float32)
    return pl.pallas_call(
        body,
        grid=(L, nb),
        in_specs=[
            in_blk,
            in_blk,
            pl.BlockSpec((EMB, EMB), lambda l, j: (0, 0)),
            pl.BlockSpec((EMB, 1), lambda l, j: (0, 0)),
        ],
        out_specs=[out_blk] * 8,
        out_shape=(out_sh,) * 8,
    )(tmp_a, tmp_b, W, b)


def kernel(tensor_day, tensor_diagnoses, table, W, b):
    B, L = tensor_day.shape
    n = B * L
    idx_diag = tensor_diagnoses.T.reshape(n).astype(jnp.int32)
    idx_day = tensor_day.T.reshape(n).astype(jnp.int32)

    table2 = _tc_transpose_table(table)
    table_lin = table2.reshape(table2.shape[0] * 2, EMB)

    tmp_diag, tmp_day = _sc_gather_pair(table_lin, idx_diag, idx_day)
    tmp_diag = tmp_diag.reshape(L, B // 512, 256, 128)
    tmp_day = tmp_day.reshape(L, B // 512, 256, 128)

    (ed_x, ed_y, ey_x, ey_y, pd_x, pd_y, py_x, py_y) = _tc_finalize(
        tmp_diag, tmp_day, W, b.reshape(EMB, 1), L, B
    )

    def to_logical(x):
        return jnp.transpose(x, (2, 0, 1))

    return (
        (to_logical(ed_x), to_logical(ey_x)),
        (to_logical(pd_x), to_logical(py_x)),
        (to_logical(ed_y), to_logical(ey_y)),
        (to_logical(pd_y), to_logical(py_y)),
    )


# speedup vs baseline: 1.2346x; 1.0140x over previous
import functools

import jax
import jax.numpy as jnp
from jax import lax
from jax.experimental import pallas as pl
from jax.experimental.pallas import tpu as pltpu
from jax.experimental.pallas import tpu_sc as plsc

EMB = 64
NC = 2
NS = 16
NW = NC * NS
CHUNK = 512
VB = 8192


def _tc_transpose_table(table):
    V = table.shape[0]
    K = VB // 2
    grid = (V + VB - 1) // VB

    def body(x_ref, y_ref):
        x = x_ref[...]
        xt_l = x[:, :K].T
        eye = (jax.lax.broadcasted_iota(jnp.int32, (EMB, EMB), 0)
               == jax.lax.broadcasted_iota(jnp.int32, (EMB, EMB), 1)
               ).astype(jnp.float32)
        xt_r = jax.lax.dot_general(
            x[:, K:], eye, (((0,), (0,)), ((), ())),
            preferred_element_type=jnp.float32,
        )
        y_ref[...] = jnp.concatenate([xt_l, xt_r], axis=1)

    return pl.pallas_call(
        body,
        grid=(grid,),
        in_specs=[pl.BlockSpec((EMB, VB), lambda j: (0, j))],
        out_specs=pl.BlockSpec((K, 2 * EMB), lambda j: (j, 0)),
        out_shape=jax.ShapeDtypeStruct((grid * K, 2 * EMB), jnp.float32),
    )(table.T)


def _sc_gather_pair(table, idx_a, idx_b):
    n = idx_a.shape[0]
    per_w = n // NW
    n_chunks = per_w // CHUNK
    assert per_w % CHUNK == 0 and per_w % 512 == 0 and n % NW == 0

    mesh = plsc.VectorSubcoreMesh(core_axis_name="c", subcore_axis_name="s")

    @functools.partial(
        pl.kernel,
        out_type=(
            jax.ShapeDtypeStruct((n, EMB), jnp.float32),
            jax.ShapeDtypeStruct((n, EMB), jnp.float32),
        ),
        mesh=mesh,
        scratch_types=[
            pltpu.VMEM((per_w,), jnp.int32),
            pltpu.VMEM((per_w,), jnp.int32),
            pltpu.VMEM((per_w,), jnp.int32),
            pltpu.VMEM((per_w,), jnp.int32),
            pltpu.VMEM((CHUNK, EMB), jnp.float32),
            pltpu.VMEM((CHUNK, EMB), jnp.float32),
            pltpu.SemaphoreType.DMA,
            pltpu.SemaphoreType.DMA,
        ],
        compiler_params=pltpu.CompilerParams(
            use_tc_tiling_on_sc=False, needs_layout_passes=False
        ),
    )
    def gather_kernel(idx_a_hbm, idx_b_hbm, table_hbm, out_a_hbm, out_b_hbm,
                      raw_a_v, raw_b_v, idx_a_v, idx_b_v, buf0, buf1,
                      sem0, sem1):
        wid = lax.axis_index("s") * NC + lax.axis_index("c")
        base = wid * per_w
        pltpu.sync_copy(idx_a_hbm.at[pl.ds(base, per_w)], raw_a_v)
        pltpu.sync_copy(idx_b_hbm.at[pl.ds(base, per_w)], raw_b_v)

        lane = lax.iota(jnp.int32, 16)

        def permute_remap(raw_v, idx_v):
            def body(k, _):
                p = k * 16 + lane
                q = p & 511
                src = (p - q) + ((q & 1) << 8) + (q >> 1)
                v = plsc.load_gather(raw_v, [src])
                r = v & (VB - 1)
                i = (v - r) + ((r & (VB // 2 - 1)) << 1) + (r >> 12)
                idx_v[pl.ds(k * 16, 16)] = i
                return ()

            lax.fori_loop(0, per_w // 16, body, (), unroll=False)

        permute_remap(raw_a_v, idx_a_v)
        permute_remap(raw_b_v, idx_b_v)

        def one_array(idx_v, out_hbm):
            def body(c, _):
                off = c * CHUNK
                pltpu.async_copy(
                    table_hbm.at[idx_v.at[pl.ds(off, CHUNK)]], buf0, sem0
                ).wait()
                pltpu.sync_copy(buf0, out_hbm.at[pl.ds(base + off, CHUNK)])
                return ()

            lax.fori_loop(0, n_chunks, body, (), unroll=False)

        one_array(idx_a_v, out_a_hbm)
        one_array(idx_b_v, out_b_hbm)

    return gather_kernel(idx_a, idx_b, table)


def _tc_finalize(tmp_a, tmp_b, W, b, L, B):
    nb = B // 4096

    def body(ta_ref, tb_ref, w_ref, b_ref,
             ea1_ref, ea2_ref, eb1_ref, eb2_ref,
             pa1_ref, pa2_ref, pb1_ref, pb2_ref):
        w = w_ref[...]
        bc = b_ref[...]

        def one(t_ref, e1_ref, e2_ref, p1_ref, p2_ref):
            halves = []
            for g in range(8):
                xt = t_ref[0, g].T
                halves.append(xt[:EMB])
                halves.append(xt[EMB:])
            e = jnp.concatenate(halves, axis=1)
            e1_ref[0] = e
            e2_ref[0] = e
            p = jnp.dot(w, jnp.maximum(e, 0.0),
                        preferred_element_type=jnp.float32) + bc
            p1_ref[0] = p
            p2_ref[0] = p

        one(ta_ref, ea1_ref, ea2_ref, pa1_ref, pa2_ref)
        one(tb_ref, eb1_ref, eb2_ref, pb1_ref, pb2_ref)

    in_blk = pl.BlockSpec((1, 8, 256, 128), lambda l, j: (l, j, 0, 0))
    out_blk = pl.BlockSpec((1, EMB, 4096), lambda l, j: (l, 0, j))
    out_sh = jax.ShapeDtypeStruct((L, EMB, B), jnp.float32)
    return pl.pallas_call(
        body,
        grid=(L, nb),
        in_specs=[
            in_blk,
            in_blk,
            pl.BlockSpec((EMB, EMB), lambda l, j: (0, 0)),
            pl.BlockSpec((EMB, 1), lambda l, j: (0, 0)),
        ],
        out_specs=[out_blk] * 8,
        out_shape=(out_sh,) * 8,
    )(tmp_a, tmp_b, W, b)


def kernel(tensor_day, tensor_diagnoses, table, W, b):
    B, L = tensor_day.shape
    n = B * L
    idx_diag = tensor_diagnoses.T.reshape(n).astype(jnp.int32)
    idx_day = tensor_day.T.reshape(n).astype(jnp.int32)

    table2 = _tc_transpose_table(table)
    table_lin = table2.reshape(table2.shape[0] * 2, EMB)

    tmp_diag, tmp_day = _sc_gather_pair(table_lin, idx_diag, idx_day)
    tmp_diag = tmp_diag.reshape(L, B // 512, 256, 128)
    tmp_day = tmp_day.reshape(L, B // 512, 256, 128)

    (ed_x, ed_y, ey_x, ey_y, pd_x, pd_y, py_x, py_y) = _tc_finalize(
        tmp_diag, tmp_day, W, b.reshape(EMB, 1), L, B
    )

    def to_logical(x):
        return jnp.transpose(x, (2, 0, 1))

    return (
        (to_logical(ed_x), to_logical(ey_x)),
        (to_logical(pd_x), to_logical(py_x)),
        (to_logical(ed_y), to_logical(ey_y)),
        (to_logical(pd_y), to_logical(py_y)),
    )


# speedup vs baseline: 1.3552x; 1.0977x over previous
import functools

import jax
import jax.numpy as jnp
from jax import lax
from jax.experimental import pallas as pl
from jax.experimental.pallas import tpu as pltpu
from jax.experimental.pallas import tpu_sc as plsc

EMB = 64
NC = 2
NS = 16
NW = NC * NS
CHUNK = 512
VB = 16384
VB_SHIFT = VB.bit_length() - 2


def _tc_transpose_table(table):
    V = table.shape[0]
    K = VB // 2
    grid = (V + VB - 1) // VB

    def body(x_ref, y_ref):
        x = x_ref[...]
        xt_l = x[:, :K].T
        eye = (jax.lax.broadcasted_iota(jnp.int32, (EMB, EMB), 0)
               == jax.lax.broadcasted_iota(jnp.int32, (EMB, EMB), 1)
               ).astype(jnp.float32)
        xt_r = jax.lax.dot_general(
            x[:, K:], eye, (((0,), (0,)), ((), ())),
            preferred_element_type=jnp.float32,
        )
        y_ref[...] = jnp.concatenate([xt_l, xt_r], axis=1)

    return pl.pallas_call(
        body,
        grid=(grid,),
        in_specs=[pl.BlockSpec((EMB, VB), lambda j: (0, j))],
        out_specs=pl.BlockSpec((K, 2 * EMB), lambda j: (j, 0)),
        out_shape=jax.ShapeDtypeStruct((grid * K, 2 * EMB), jnp.float32),
    )(table.T)


def _sc_gather_pair(table, idx_a, idx_b):
    n = idx_a.shape[0]
    per_w = n // NW
    n_chunks = per_w // CHUNK
    assert per_w % CHUNK == 0 and per_w % 512 == 0 and n % NW == 0

    mesh = plsc.VectorSubcoreMesh(core_axis_name="c", subcore_axis_name="s")

    @functools.partial(
        pl.kernel,
        out_type=(
            jax.ShapeDtypeStruct((n, EMB), jnp.float32),
            jax.ShapeDtypeStruct((n, EMB), jnp.float32),
        ),
        mesh=mesh,
        scratch_types=[
            pltpu.VMEM((per_w,), jnp.int32),
            pltpu.VMEM((per_w,), jnp.int32),
            pltpu.VMEM((per_w,), jnp.int32),
            pltpu.VMEM((per_w,), jnp.int32),
            pltpu.VMEM((CHUNK, EMB), jnp.float32),
            pltpu.VMEM((CHUNK, EMB), jnp.float32),
            pltpu.SemaphoreType.DMA,
            pltpu.SemaphoreType.DMA,
        ],
        compiler_params=pltpu.CompilerParams(
            use_tc_tiling_on_sc=False, needs_layout_passes=False
        ),
    )
    def gather_kernel(idx_a_hbm, idx_b_hbm, table_hbm, out_a_hbm, out_b_hbm,
                      raw_a_v, raw_b_v, idx_a_v, idx_b_v, buf0, buf1,
                      sem0, sem1):
        wid = lax.axis_index("s") * NC + lax.axis_index("c")
        base = wid * per_w
        pltpu.sync_copy(idx_a_hbm.at[pl.ds(base, per_w)], raw_a_v)
        pltpu.sync_copy(idx_b_hbm.at[pl.ds(base, per_w)], raw_b_v)

        lane = lax.iota(jnp.int32, 16)

        def permute_remap(raw_v, idx_v):
            def body(k, _):
                p = k * 16 + lane
                q = p & 511
                src = (p - q) + ((q & 1) << 8) + (q >> 1)
                v = plsc.load_gather(raw_v, [src])
                r = v & (VB - 1)
                i = (v - r) + ((r & (VB // 2 - 1)) << 1) + (r >> VB_SHIFT)
                idx_v[pl.ds(k * 16, 16)] = i
                return ()

            lax.fori_loop(0, per_w // 16, body, (), unroll=False)

        permute_remap(raw_a_v, idx_a_v)
        permute_remap(raw_b_v, idx_b_v)

        def gather_start(idx_v, c, buf, sem):
            pltpu.async_copy(
                table_hbm.at[idx_v.at[pl.ds(c * CHUNK, CHUNK)]], buf, sem
            )

        def gather_wait(buf, sem):
            pltpu.make_async_copy(
                table_hbm.at[pl.ds(0, CHUNK)], buf, sem
            ).wait()

        def one_array(idx_v, out_hbm):
            last = n_chunks - 1
            gather_start(idx_v, 0, buf0, sem0)

            def body(h, _):
                c0 = h * 2
                c1 = c0 + 1
                c2 = jnp.minimum(c0 + 2, last)
                gather_wait(buf0, sem0)
                gather_start(idx_v, c1, buf1, sem1)
                pltpu.sync_copy(
                    buf0, out_hbm.at[pl.ds(base + c0 * CHUNK, CHUNK)]
                )
                gather_wait(buf1, sem1)
                gather_start(idx_v, c2, buf0, sem0)
                pltpu.sync_copy(
                    buf1, out_hbm.at[pl.ds(base + c1 * CHUNK, CHUNK)]
                )
                return ()

            lax.fori_loop(0, n_chunks // 2, body, (), unroll=False)
            gather_wait(buf0, sem0)
            pltpu.sync_copy(
                buf0, out_hbm.at[pl.ds(base + last * CHUNK, CHUNK)]
            )

        one_array(idx_a_v, out_a_hbm)
        one_array(idx_b_v, out_b_hbm)

    return gather_kernel(idx_a, idx_b, table)


def _tc_finalize(tmp_a, tmp_b, W, b, L, B):
    nb = B // 4096

    def body(ta_ref, tb_ref, w_ref, b_ref,
             ea1_ref, ea2_ref, eb1_ref, eb2_ref,
             pa1_ref, pa2_ref, pb1_ref, pb2_ref):
        w = w_ref[...]
        bc = b_ref[...]

        def one(t_ref, e1_ref, e2_ref, p1_ref, p2_ref):
            halves = []
            for g in range(8):
                xt = t_ref[0, g].T
                halves.append(xt[:EMB])
                halves.append(xt[EMB:])
            e = jnp.concatenate(halves, axis=1)
            e1_ref[0] = e
            e2_ref[0] = e
            p = jnp.dot(w, jnp.maximum(e, 0.0),
                        preferred_element_type=jnp.float32) + bc
            p1_ref[0] = p
            p2_ref[0] = p

        one(ta_ref, ea1_ref, ea2_ref, pa1_ref, pa2_ref)
        one(tb_ref, eb1_ref, eb2_ref, pb1_ref, pb2_ref)

    in_blk = pl.BlockSpec((1, 8, 256, 128), lambda l, j: (l, j, 0, 0))
    out_blk = pl.BlockSpec((1, EMB, 4096), lambda l, j: (l, 0, j))
    out_sh = jax.ShapeDtypeStruct((L, EMB, B), jnp.float32)
    return pl.pallas_call(
        body,
        grid=(L, nb),
        in_specs=[
            in_blk,
            in_blk,
            pl.BlockSpec((EMB, EMB), lambda l, j: (0, 0)),
            pl.BlockSpec((EMB, 1), lambda l, j: (0, 0)),
        ],
        out_specs=[out_blk] * 8,
        out_shape=(out_sh,) * 8,
    )(tmp_a, tmp_b, W, b)


def kernel(tensor_day, tensor_diagnoses, table, W, b):
    B, L = tensor_day.shape
    n = B * L
    idx_diag = tensor_diagnoses.T.reshape(n).astype(jnp.int32)
    idx_day = tensor_day.T.reshape(n).astype(jnp.int32)

    table2 = _tc_transpose_table(table)
    table_lin = table2.reshape(table2.shape[0] * 2, EMB)

    tmp_diag, tmp_day = _sc_gather_pair(table_lin, idx_diag, idx_day)
    tmp_diag = tmp_diag.reshape(L, B // 512, 256, 128)
    tmp_day = tmp_day.reshape(L, B // 512, 256, 128)

    (ed_x, ed_y, ey_x, ey_y, pd_x, pd_y, py_x, py_y) = _tc_finalize(
        tmp_diag, tmp_day, W, b.reshape(EMB, 1), L, B
    )

    def to_logical(x):
        return jnp.transpose(x, (2, 0, 1))

    return (
        (to_logical(ed_x), to_logical(ey_x)),
        (to_logical(pd_x), to_logical(py_x)),
        (to_logical(ed_y), to_logical(ey_y)),
        (to_logical(pd_y), to_logical(py_y)),
    )
